# Initial kernel scaffold; baseline (speedup 1.0000x reference)
import jax
import jax.numpy as jnp
from jax.experimental import pallas as pl


def kernel(x, edge_index, W, b, prelu_a):
    raise NotImplementedError("write your pallas kernel here")


# speedup vs baseline: 10.6387x; 10.6387x over previous
import functools

import jax
import jax.numpy as jnp
from jax import lax
from jax.experimental import pallas as pl
from jax.experimental.pallas import tpu as pltpu
from jax.experimental.pallas import tpu_sc as plsc

D = 128
NC = 2
NS = 16
EB = 128
TR = 104
WB = 200


def _zero_acc(s, half, zeros_hbm, zero_v, acc_sh):
  zrows = (half + TR) // NS
  pltpu.sync_copy(zeros_hbm, zero_v)
  z0 = s * zrows
  for step in range(0, zrows, EB):
    blk = min(EB, zrows - step)
    pltpu.sync_copy(zero_v.at[pl.ds(0, blk)], acc_sh.at[pl.ds(z0 + step, blk)])


def _write_acc(c, s, half, out, acc_sh):
  wchunks = half // WB
  for k in range((wchunks + NS - 1) // NS):
    cid = s + NS * k

    @pl.when(cid < wchunks)
    def _():
      pltpu.sync_copy(acc_sh.at[pl.ds(cid * WB, WB)],
                      out.at[c, pl.ds(cid * WB, WB)])


def _deg_body(chunks, half, rdstr, ones_hbm, zeros_hbm, out,
              dst_v, ones_v, zero_v, acc_sh):
  c = lax.axis_index("c")
  s = lax.axis_index("s")

  _zero_acc(s, half, zeros_hbm, zero_v, acc_sh)
  pltpu.sync_copy(ones_hbm, ones_v)
  plsc.subcore_barrier()
  pltpu.sync_copy(rdstr.at[c, s], dst_v)

  @pl.loop(0, chunks)
  def _(j):
    pltpu.sync_copy(ones_v, acc_sh.at[dst_v.at[j]], add=True)

  plsc.subcore_barrier()
  _write_acc(c, s, half, out, acc_sh)


def _mp_body(chunks, half, yw, srcr, rdstr, zeros_hbm, out,
             src_v, dst_v, rows_v, zero_v, sem0, sem1, acc_sh):
  c = lax.axis_index("c")
  s = lax.axis_index("s")

  _zero_acc(s, half, zeros_hbm, zero_v, acc_sh)
  plsc.subcore_barrier()
  pltpu.sync_copy(srcr.at[s], src_v)
  pltpu.sync_copy(rdstr.at[c, s], dst_v)

  sems = (sem0, sem1)
  pltpu.make_async_copy(yw.at[src_v.at[0]], rows_v.at[0], sem0).start()

  @pl.loop(0, chunks, step=2)
  def _(j2):
    for b in range(2):
      j = j2 + b
      nb = 1 - b

      @pl.when(j + 1 < chunks)
      def _():
        pltpu.make_async_copy(yw.at[src_v.at[j + 1]], rows_v.at[nb],
                              sems[nb]).start()

      pltpu.make_async_copy(yw.at[src_v.at[j]], rows_v.at[b], sems[b]).wait()
      pltpu.sync_copy(rows_v.at[b], acc_sh.at[dst_v.at[j]], add=True)

  plsc.subcore_barrier()
  _write_acc(c, s, half, out, acc_sh)


def _remap_body(half, dst_ref, rd_ref):
  s = pl.program_id(0)
  d = dst_ref[0]
  for c in range(NC):
    loc = d - c * half
    m = (loc >= 0) & (loc < half)
    rd_ref[c, 0] = jnp.where(m, loc, half + 4 * s)


def _tc1_body(x_ref, w_ref, dm_ref, yw_ref):
  deg = dm_ref[:, 0:1] + 1.0
  dinv = lax.rsqrt(deg)
  xw = jnp.dot(x_ref[...], w_ref[...], preferred_element_type=jnp.float32)
  yw_ref[...] = xw * dinv


def _tc2_body(p_ref, yw_ref, dm_ref, b_ref, a_ref, o_ref):
  deg = dm_ref[:, 0:1] + 1.0
  dinv = lax.rsqrt(deg)
  t = (p_ref[...] + yw_ref[...]) * dinv + b_ref[...]
  o_ref[...] = jnp.where(t >= 0, t, t * a_ref[...])


def kernel(x, edge_index, W, b, prelu_a):
  n = x.shape[0]
  e = edge_index.shape[1]
  half = ((n + NC - 1) // NC + 7) // 8 * 8
  chunks = (e + NS * EB - 1) // (NS * EB)
  chunks += chunks % 2
  e_pad = NS * chunks * EB

  src = edge_index[0].astype(jnp.int32)
  dst = edge_index[1].astype(jnp.int32)
  spad = jnp.zeros((e_pad - e,), dtype=jnp.int32)
  dpad = jnp.full((e_pad - e,), -1, dtype=jnp.int32)
  srcr = jnp.concatenate([src, spad]).reshape(NS, chunks, EB)
  dstr = jnp.concatenate([dst, dpad]).reshape(NS, chunks, EB)

  mesh = plsc.VectorSubcoreMesh(core_axis_name="c", subcore_axis_name="s",
                                num_cores=NC, num_subcores=NS)

  rdstr = pl.pallas_call(
      functools.partial(_remap_body, half),
      grid=(NS,),
      in_specs=[pl.BlockSpec((1, chunks, EB), lambda s: (s, 0, 0))],
      out_specs=pl.BlockSpec((NC, 1, chunks, EB), lambda s: (0, s, 0, 0)),
      out_shape=jax.ShapeDtypeStruct((NC, NS, chunks, EB), jnp.int32),
  )(dstr)

  ones_big = jnp.ones((EB, D), jnp.float32)
  zeros_big = jnp.zeros((EB, D), jnp.float32)

  deg_k = pl.kernel(
      functools.partial(_deg_body, chunks, half),
      out_type=jax.ShapeDtypeStruct((NC, half, D), jnp.float32),
      mesh=mesh,
      scratch_types=[
          pltpu.VMEM((chunks, EB), jnp.int32),
          pltpu.VMEM((EB, D), jnp.float32),
          pltpu.VMEM((EB, D), jnp.float32),
          pltpu.VMEM_SHARED((half + TR, D), jnp.float32),
      ],
  )
  deg_mat = deg_k(rdstr, ones_big, zeros_big).reshape(NC * half, D)

  rb = 1000
  grid = n // rb
  yw = pl.pallas_call(
      _tc1_body,
      grid=(grid,),
      in_specs=[
          pl.BlockSpec((rb, D), lambda i: (i, 0)),
          pl.BlockSpec((D, D), lambda i: (0, 0)),
          pl.BlockSpec((rb, D), lambda i: (i, 0)),
      ],
      out_specs=pl.BlockSpec((rb, D), lambda i: (i, 0)),
      out_shape=jax.ShapeDtypeStruct((n, D), jnp.float32),
  )(x, W, deg_mat)

  mp = pl.kernel(
      functools.partial(_mp_body, chunks, half),
      out_type=jax.ShapeDtypeStruct((NC, half, D), jnp.float32),
      mesh=mesh,
      scratch_types=[
          pltpu.VMEM((chunks, EB), jnp.int32),
          pltpu.VMEM((chunks, EB), jnp.int32),
          pltpu.VMEM((2, EB, D), jnp.float32),
          pltpu.VMEM((EB, D), jnp.float32),
          pltpu.SemaphoreType.DMA,
          pltpu.SemaphoreType.DMA,
          pltpu.VMEM_SHARED((half + TR, D), jnp.float32),
      ],
  )
  p = mp(yw, srcr, rdstr, zeros_big).reshape(NC * half, D)

  out = pl.pallas_call(
      _tc2_body,
      grid=(grid,),
      in_specs=[
          pl.BlockSpec((rb, D), lambda i: (i, 0)),
          pl.BlockSpec((rb, D), lambda i: (i, 0)),
          pl.BlockSpec((rb, D), lambda i: (i, 0)),
          pl.BlockSpec((D,), lambda i: (0,)),
          pl.BlockSpec((D,), lambda i: (0,)),
      ],
      out_specs=pl.BlockSpec((rb, D), lambda i: (i, 0)),
      out_shape=jax.ShapeDtypeStruct((n, D), jnp.float32),
  )(p, yw, deg_mat, b, prelu_a)

  return out
